# Initial kernel scaffold; baseline (speedup 1.0000x reference)
import jax
import jax.numpy as jnp
from jax.experimental import pallas as pl


def kernel(camera_features, uu, vv, dd, valid):
    raise NotImplementedError("write your pallas kernel here")


# speedup vs baseline: 1868.8704x; 1868.8704x over previous
import functools

import jax
import jax.numpy as jnp
from jax import lax
from jax.experimental import pallas as pl
from jax.experimental.pallas import tpu as pltpu
from jax.experimental.pallas import tpu_sc as plsc

VOXEL_SHAPE = (4, 200, 200)
V = VOXEL_SHAPE[0] * VOXEL_SHAPE[1] * VOXEL_SHAPE[2]
NCAM = 6
C = 32
D, H, W = 32, 32, 60
DHW = D * H * W

NC, NS, L = 2, 16, 16
NW = NC * NS
CH = 512
CPW = 10
VP = NW * CH * CPW
NBUF = 2
GROUPS = CH // L
IGNORE = -1

_mesh = plsc.VectorSubcoreMesh(
    core_axis_name="c", subcore_axis_name="s", num_cores=NC, num_subcores=NS
)


@functools.partial(
    pl.kernel,
    out_type=jax.ShapeDtypeStruct((VP, C), jnp.float32),
    mesh=_mesh,
    scratch_types=(
        [pltpu.VMEM((NCAM, CH), jnp.int32) for _ in range(4 * NBUF)]
        + [pltpu.VMEM((CH,), jnp.int32) for _ in range(NCAM * NBUF)]
        + [pltpu.VMEM((CH, C), jnp.float32) for _ in range(NBUF)]
        + [pltpu.SemaphoreType.DMA for _ in range(2 * NBUF)]
    ),
    compiler_params=pltpu.CompilerParams(use_tc_tiling_on_sc=False),
)
def _fastray_sc(table, uu, vv, dd, va, out, *scratch):
    uub = scratch[0:NBUF]
    vvb = scratch[NBUF : 2 * NBUF]
    ddb = scratch[2 * NBUF : 3 * NBUF]
    vab = scratch[3 * NBUF : 4 * NBUF]
    k = 4 * NBUF
    idxb = [scratch[k + b * NCAM : k + (b + 1) * NCAM] for b in range(NBUF)]
    k += NCAM * NBUF
    accb = scratch[k : k + NBUF]
    gsem = scratch[k + NBUF : k + 2 * NBUF]
    ssem = scratch[k + 2 * NBUF : k + 3 * NBUF]

    wid = lax.axis_index("s") * NC + lax.axis_index("c")
    base0 = wid * (CH * CPW)

    def super_body(i, carry):
        bases = [base0 + (i * NBUF + b) * CH for b in range(NBUF)]
        for b in range(NBUF):
            sl = pl.ds(bases[b], CH)
            pltpu.sync_copy(uu.at[:, sl], uub[b])
            pltpu.sync_copy(vv.at[:, sl], vvb[b])
            pltpu.sync_copy(dd.at[:, sl], ddb[b])
            pltpu.sync_copy(va.at[:, sl], vab[b])

            def grp(g, _, b=b):
                off = pl.ds(g * L, L)
                for n in range(NCAM):
                    lin = (
                        ddb[b][n, off] * (H * W)
                        + vvb[b][n, off] * W
                        + uub[b][n, off]
                        + n * DHW
                    )
                    idxb[b][n][off] = jnp.where(
                        vab[b][n, off] != 0, lin, IGNORE
                    )
                return 0

            lax.fori_loop(0, GROUPS, grp, 0)

            @pl.when(i > 0)
            def _():
                pltpu.make_async_copy(
                    accb[b], out.at[pl.ds(0, CH), :], ssem[b]
                ).wait()

            def zrow(r, _, b=b):
                accb[b][r, pl.ds(0, L)] = jnp.zeros((L,), jnp.float32)
                accb[b][r, pl.ds(L, L)] = jnp.zeros((L,), jnp.float32)
                return 0

            lax.fori_loop(0, CH, zrow, 0)

        handles = [[None] * NBUF for _ in range(NCAM)]
        for n in range(NCAM):
            for b in range(NBUF):
                if n > 0:
                    handles[n - 1][b].wait()
                handles[n][b] = pltpu.async_copy(
                    table.at[plsc.Indices(idxb[b][n], ignored_value=IGNORE)],
                    accb[b],
                    gsem[b],
                    add=True,
                )
        for b in range(NBUF):
            handles[NCAM - 1][b].wait()
            pltpu.async_copy(accb[b], out.at[pl.ds(bases[b], CH), :], ssem[b])
        return carry

    lax.fori_loop(0, CPW // NBUF, super_body, 0)
    for b in range(NBUF):
        pltpu.make_async_copy(accb[b], out.at[pl.ds(0, CH), :], ssem[b]).wait()


def kernel(camera_features, uu, vv, dd, valid):
    B = camera_features.shape[0]
    feat = camera_features.reshape(NCAM, C, DHW)
    table = jnp.swapaxes(feat, 1, 2).reshape(NCAM * DHW, C)
    pad = ((0, 0), (0, VP - V))
    uu_p = jnp.pad(uu, pad)
    vv_p = jnp.pad(vv, pad)
    dd_p = jnp.pad(dd, pad)
    va_p = jnp.pad(valid.astype(jnp.int32), pad)
    out_t = _fastray_sc(table, uu_p, vv_p, dd_p, va_p)
    return out_t[:V].T.reshape(B, C, *VOXEL_SHAPE)
